# Initial kernel scaffold; baseline (speedup 1.0000x reference)
import jax
import jax.numpy as jnp
from jax.experimental import pallas as pl


def kernel(u, i, j, edge_index, edge_vals, user_emb, item_emb, W1_w, W1_b, W2_w, W2_b):
    raise NotImplementedError("write your pallas kernel here")


# speedup vs baseline: 1.0074x; 1.0074x over previous
import functools

import jax
import jax.numpy as jnp
from jax.experimental import pallas as pl
from jax.experimental.pallas import tpu as pltpu

N_USERS = 25000
N_ITEMS = 25000
N = N_USERS + N_ITEMS
D = 64
L = 3
E = 800000
B = 4096

_BLK = 2000


def _dense_body(emb_ref, sp_ref, w1_ref, b1_ref, w2_ref, b2_ref, new_ref, norm_ref):
    e = emb_ref[...]
    s = sp_ref[...]
    t1 = jax.lax.dot_general(s, w1_ref[...], (((1,), (1,)), ((), ())),
                             preferred_element_type=jnp.float32) + b1_ref[...]
    t2 = jax.lax.dot_general(e * s, w2_ref[...], (((1,), (1,)), ((), ())),
                             preferred_element_type=jnp.float32) + b2_ref[...]
    x = t1 + t2
    x = jnp.where(x > 0, x, 0.01 * x)
    n = jnp.sqrt(jnp.sum(x * x, axis=1, keepdims=True))
    norm_ref[...] = x / jnp.maximum(n, 1e-12)
    new_ref[...] = x


def _dense_layer(emb, sp, w1, b1, w2, b2):
    grid = (N // _BLK,)
    return pl.pallas_call(
        _dense_body,
        grid=grid,
        in_specs=[
            pl.BlockSpec((_BLK, D), lambda i: (i, 0)),
            pl.BlockSpec((_BLK, D), lambda i: (i, 0)),
            pl.BlockSpec((D, D), lambda i: (0, 0)),
            pl.BlockSpec((1, D), lambda i: (0, 0)),
            pl.BlockSpec((D, D), lambda i: (0, 0)),
            pl.BlockSpec((1, D), lambda i: (0, 0)),
        ],
        out_specs=[
            pl.BlockSpec((_BLK, D), lambda i: (i, 0)),
            pl.BlockSpec((_BLK, D), lambda i: (i, 0)),
        ],
        out_shape=[
            jax.ShapeDtypeStruct((N, D), jnp.float32),
            jax.ShapeDtypeStruct((N, D), jnp.float32),
        ],
    )(emb, sp, w1, b1.reshape(1, D), w2, b2.reshape(1, D))


def _loss_body(bu_ref, bp_ref, bn_ref, out_ref):
    bu = bu_ref[...]
    pos = jnp.sum(bu * bp_ref[...], axis=1)
    neg = jnp.sum(bu * bn_ref[...], axis=1)
    x = pos - neg
    out_ref[0, 0] = -jnp.mean(jnp.log(jax.nn.sigmoid(x)))


def _loss(bu, bp, bn):
    return pl.pallas_call(
        _loss_body,
        out_shape=jax.ShapeDtypeStruct((1, 1), jnp.float32),
        out_specs=pl.BlockSpec(memory_space=pltpu.SMEM),
    )(bu, bp, bn)[0, 0]


def kernel(u, i, j, edge_index, edge_vals, user_emb, item_emb, W1_w, W1_b, W2_w, W2_b):
    emb = jnp.concatenate((user_emb, item_emb), axis=0)
    src = edge_index[1].astype(jnp.int32)
    dst = edge_index[0].astype(jnp.int32)

    finals = [emb]
    for l in range(L):
        msgs = edge_vals[:, None] * jnp.take(emb, src, axis=0)
        sp = jax.ops.segment_sum(msgs, dst, num_segments=N)
        emb, norm = _dense_layer(emb, sp, W1_w[l], W1_b[l], W2_w[l], W2_b[l])
        finals.append(norm)

    final = jnp.concatenate(finals, axis=1)
    bu = jnp.take(final[:N_USERS], u, axis=0)
    bp = jnp.take(final[N_USERS:], i, axis=0)
    bn = jnp.take(final[N_USERS:], j, axis=0)
    return _loss(bu, bp, bn)


# speedup vs baseline: 1.8827x; 1.8688x over previous
import functools

import jax
import jax.numpy as jnp
from jax import lax
from jax.experimental import pallas as pl
from jax.experimental.pallas import tpu as pltpu
from jax.experimental.pallas import tpu_sc as plsc

N_USERS = 25000
N_ITEMS = 25000
N = N_USERS + N_ITEMS
D = 64
L = 3
E = 800000
B = 4096

_BLK = 2000

_NS = 16
_G = 2
_EPAD = 819200
_EROWS = _EPAD // 128
_RPT = _EROWS // _NS
_NCHUNK = _RPT // _G
_HALF = 25088
_DUMMY = _HALF
_ACC_ROWS = 25600
_ZCH = 64
_OPT_ROWS = _HALF // _NS


def _spmm_body(src_hbm, dst_hbm, val_hbm, emb_hbm, out_hbm,
               idx_src, idx_dst, vals_v, rows_v, zbuf, acc, gsem, ssem):
    cid = lax.axis_index("c")
    sid = lax.axis_index("s")

    zero16 = jnp.zeros((16,), jnp.float32)

    @pl.loop(0, _ZCH)
    def _zero_zbuf(k):
        for m in range(4):
            zbuf[k, pl.ds(m * 16, 16)] = zero16

    @pl.loop(0, 25)
    def _zero_acc(z):
        pltpu.sync_copy(zbuf, acc.at[pl.ds((sid * 25 + z) * _ZCH, _ZCH)])

    plsc.subcore_barrier()

    @pl.loop(0, _NCHUNK)
    def _chunk(c):
        base = sid * _RPT + c * _G
        pltpu.sync_copy(src_hbm.at[pl.ds(base, _G)], idx_src)
        pltpu.sync_copy(dst_hbm.at[pl.ds(base, _G)], idx_dst)
        pltpu.sync_copy(val_hbm.at[pl.ds(base * 128, _G * 128)], vals_v)

        gathers = [
            pltpu.async_copy(emb_hbm.at[idx_src.at[g]],
                             rows_v.at[pl.ds(g * 128, 128)], gsem)
            for g in range(_G)
        ]
        for cp in gathers:
            cp.wait()

        @pl.loop(0, _G * 128 // 16)
        def _scale(p):
            vgrp = vals_v[pl.ds(p * 16, 16)]

            for q in range(16):
                v = vgrp[q]
                e = p * 16 + q
                for m in range(4):
                    sl = pl.ds(m * 16, 16)
                    rows_v[e, sl] = rows_v[e, sl] * v

        off = cid * _HALF
        for g in range(_G):
            for p in range(8):
                sl = pl.ds(p * 16, 16)
                d = idx_dst[g, sl] - off
                ok = (d >= 0) & (d < _HALF)
                idx_dst[g, sl] = jnp.where(ok, d, _DUMMY)

        scatters = [
            pltpu.async_copy(rows_v.at[pl.ds(g * 128, 128)],
                             acc.at[idx_dst.at[g]], ssem, add=True)
            for g in range(_G)
        ]
        for cp in scatters:
            cp.wait()

    plsc.subcore_barrier()
    pltpu.sync_copy(acc.at[pl.ds(sid * _OPT_ROWS, _OPT_ROWS)],
                    out_hbm.at[pl.ds(cid * _HALF + sid * _OPT_ROWS, _OPT_ROWS)])


_spmm_call = functools.partial(
    pl.kernel,
    out_type=jax.ShapeDtypeStruct((2 * _HALF, D), jnp.float32),
    mesh=plsc.VectorSubcoreMesh(core_axis_name="c", subcore_axis_name="s"),
    scratch_types=[
        pltpu.VMEM((_G, 128), jnp.int32),
        pltpu.VMEM((_G, 128), jnp.int32),
        pltpu.VMEM((_G * 128,), jnp.float32),
        pltpu.VMEM((_G * 128, D), jnp.float32),
        pltpu.VMEM((_ZCH, D), jnp.float32),
        pltpu.VMEM_SHARED((_ACC_ROWS, D), jnp.float32),
        pltpu.SemaphoreType.DMA,
        pltpu.SemaphoreType.DMA,
    ],
    compiler_params=pltpu.CompilerParams(use_tc_tiling_on_sc=False),
)(_spmm_body)


def _dense_body(emb_ref, sp_ref, w1_ref, b1_ref, w2_ref, b2_ref, new_ref, norm_ref):
    e = emb_ref[...]
    s = sp_ref[...]
    t1 = jax.lax.dot_general(s, w1_ref[...], (((1,), (1,)), ((), ())),
                             preferred_element_type=jnp.float32) + b1_ref[...]
    t2 = jax.lax.dot_general(e * s, w2_ref[...], (((1,), (1,)), ((), ())),
                             preferred_element_type=jnp.float32) + b2_ref[...]
    x = t1 + t2
    x = jnp.where(x > 0, x, 0.01 * x)
    n = jnp.sqrt(jnp.sum(x * x, axis=1, keepdims=True))
    norm_ref[...] = x / jnp.maximum(n, 1e-12)
    new_ref[...] = x


def _dense_layer(emb, sp, w1, b1, w2, b2):
    grid = (N // _BLK,)
    return pl.pallas_call(
        _dense_body,
        grid=grid,
        in_specs=[
            pl.BlockSpec((_BLK, D), lambda i: (i, 0)),
            pl.BlockSpec((_BLK, D), lambda i: (i, 0)),
            pl.BlockSpec((D, D), lambda i: (0, 0)),
            pl.BlockSpec((1, D), lambda i: (0, 0)),
            pl.BlockSpec((D, D), lambda i: (0, 0)),
            pl.BlockSpec((1, D), lambda i: (0, 0)),
        ],
        out_specs=[
            pl.BlockSpec((_BLK, D), lambda i: (i, 0)),
            pl.BlockSpec((_BLK, D), lambda i: (i, 0)),
        ],
        out_shape=[
            jax.ShapeDtypeStruct((N, D), jnp.float32),
            jax.ShapeDtypeStruct((N, D), jnp.float32),
        ],
    )(emb, sp, w1, b1.reshape(1, D), w2, b2.reshape(1, D))


def _loss_body(bu_ref, bp_ref, bn_ref, out_ref):
    bu = bu_ref[...]
    pos = jnp.sum(bu * bp_ref[...], axis=1)
    neg = jnp.sum(bu * bn_ref[...], axis=1)
    x = pos - neg
    out_ref[0, 0] = -jnp.mean(jnp.log(jax.nn.sigmoid(x)))


def _loss(bu, bp, bn):
    return pl.pallas_call(
        _loss_body,
        out_shape=jax.ShapeDtypeStruct((1, 1), jnp.float32),
        out_specs=pl.BlockSpec(memory_space=pltpu.SMEM),
    )(bu, bp, bn)[0, 0]


def kernel(u, i, j, edge_index, edge_vals, user_emb, item_emb, W1_w, W1_b, W2_w, W2_b):
    emb = jnp.concatenate((user_emb, item_emb), axis=0)
    pad = _EPAD - E
    src = jnp.concatenate([edge_index[1].astype(jnp.int32),
                           jnp.zeros((pad,), jnp.int32)]).reshape(_EROWS, 128)
    dst = jnp.concatenate([edge_index[0].astype(jnp.int32),
                           jnp.zeros((pad,), jnp.int32)]).reshape(_EROWS, 128)
    val = jnp.concatenate([edge_vals.astype(jnp.float32),
                           jnp.zeros((pad,), jnp.float32)])

    finals = [emb]
    for l in range(L):
        sp = _spmm_call(src, dst, val, emb)[:N]
        emb, norm = _dense_layer(emb, sp, W1_w[l], W1_b[l], W2_w[l], W2_b[l])
        finals.append(norm)

    final = jnp.concatenate(finals, axis=1)
    bu = jnp.take(final[:N_USERS], u, axis=0)
    bp = jnp.take(final[N_USERS:], i, axis=0)
    bn = jnp.take(final[N_USERS:], j, axis=0)
    return _loss(bu, bp, bn)


# speedup vs baseline: 5.3998x; 2.8681x over previous
import functools

import jax
import jax.numpy as jnp
from jax import lax
from jax.experimental import pallas as pl
from jax.experimental.pallas import tpu as pltpu
from jax.experimental.pallas import tpu_sc as plsc

N_USERS = 25000
N_ITEMS = 25000
N = N_USERS + N_ITEMS
D = 64
H = D // 2
L = 3
E = 800000
B = 4096

_BLK = 2000

_NS = 16
_NB = 4
_EPAD = 819200
_EROWS = _EPAD // 128
_RPT = _EROWS // _NS
_OROWS = 50176
_STRIPE = _OROWS // _NS


def _spmm_body(src_hbm, dst_hbm, val_hbm, emb_hbm, out_hbm,
               sb0, sb1, sb2, sb3, db0, db1, db2, db3,
               vb0, vb1, vb2, vb3, rb0, rb1, rb2, rb3,
               acc, isem, gsem, ssem):
    cid = lax.axis_index("c")
    sid = lax.axis_index("s")
    sbs = (sb0, sb1, sb2, sb3)
    dbs = (db0, db1, db2, db3)
    vbs = (vb0, vb1, vb2, vb3)
    rbs = (rb0, rb1, rb2, rb3)

    zero16 = jnp.zeros((16,), jnp.float32)

    @pl.loop(0, 64)
    def _zero_buf(k):
        rb0[k, pl.ds(0, 16)] = zero16
        rb0[k, pl.ds(16, 16)] = zero16

    @pl.loop(0, _STRIPE // 64)
    def _zero_acc(z):
        pltpu.sync_copy(rb0.at[pl.ds(0, 64)],
                        acc.at[pl.ds(sid * _STRIPE + z * 64, 64)])

    plsc.subcore_barrier()

    off = cid * N

    @pl.loop(0, _RPT // _NB)
    def _outer(o):
        s0 = sid * _RPT + o * _NB
        cps = []
        for b in range(_NB):
            row = s0 + b
            cps.append(pltpu.async_copy(src_hbm.at[pl.ds(row, 1)], sbs[b], isem))
            cps.append(pltpu.async_copy(dst_hbm.at[pl.ds(row, 1)], dbs[b], isem))
            cps.append(pltpu.async_copy(val_hbm.at[pl.ds(row * 128, 128)],
                                        vbs[b], isem))
        for cp in cps:
            cp.wait()

        gcps = []
        for b in range(_NB):
            for p in range(8):
                sl = pl.ds(p * 16, 16)
                sbs[b][0, sl] = sbs[b][0, sl] + off
            gcps.append(pltpu.async_copy(emb_hbm.at[sbs[b].at[0]], rbs[b], gsem))

        scps = []
        for b in range(_NB):
            gcps[b].wait()

            @pl.loop(0, 8)
            def _grp(p, b=b):
                vgrp = vbs[b][pl.ds(p * 16, 16)]
                for q in range(16):
                    v = vgrp[q]
                    e = p * 16 + q
                    rbs[b][e, pl.ds(0, 16)] = rbs[b][e, pl.ds(0, 16)] * v
                    rbs[b][e, pl.ds(16, 16)] = rbs[b][e, pl.ds(16, 16)] * v

            scps.append(pltpu.async_copy(rbs[b], acc.at[dbs[b].at[0]],
                                         ssem, add=True))
        for cp in scps:
            cp.wait()

    plsc.subcore_barrier()
    pltpu.sync_copy(acc.at[pl.ds(sid * _STRIPE, _STRIPE)],
                    out_hbm.at[cid, pl.ds(sid * _STRIPE, _STRIPE)])


_idx_buf = lambda: pltpu.VMEM((1, 128), jnp.int32)
_spmm_call = functools.partial(
    pl.kernel,
    out_type=jax.ShapeDtypeStruct((2, _OROWS, H), jnp.float32),
    mesh=plsc.VectorSubcoreMesh(core_axis_name="c", subcore_axis_name="s"),
    scratch_types=(
        [_idx_buf() for _ in range(2 * _NB)]
        + [pltpu.VMEM((128,), jnp.float32) for _ in range(_NB)]
        + [pltpu.VMEM((128, H), jnp.float32) for _ in range(_NB)]
        + [pltpu.VMEM_SHARED((_OROWS, H), jnp.float32),
           pltpu.SemaphoreType.DMA,
           pltpu.SemaphoreType.DMA,
           pltpu.SemaphoreType.DMA]
    ),
    compiler_params=pltpu.CompilerParams(use_tc_tiling_on_sc=False),
)(_spmm_body)


def _dense_body(e2_ref, sp_ref, w1l_ref, w1h_ref, b1_ref,
                w2l_ref, w2h_ref, b2_ref, norm_ref, e2o_ref):
    el = e2_ref[0]
    eh = e2_ref[1]
    sl_ = sp_ref[0]
    sh_ = sp_ref[1]
    dim = (((1,), (1,)), ((), ()))
    t1 = (lax.dot_general(sl_, w1l_ref[...], dim, preferred_element_type=jnp.float32)
          + lax.dot_general(sh_, w1h_ref[...], dim, preferred_element_type=jnp.float32)
          + b1_ref[...])
    t2 = (lax.dot_general(el * sl_, w2l_ref[...], dim, preferred_element_type=jnp.float32)
          + lax.dot_general(eh * sh_, w2h_ref[...], dim, preferred_element_type=jnp.float32)
          + b2_ref[...])
    x = t1 + t2
    x = jnp.where(x > 0, x, 0.01 * x)
    n = jnp.sqrt(jnp.sum(x * x, axis=1, keepdims=True))
    norm_ref[...] = x / jnp.maximum(n, 1e-12)
    e2o_ref[0] = x[:, :H]
    e2o_ref[1] = x[:, H:]


def _dense_layer(e2, sp2, w1, b1, w2, b2):
    grid = (N // _BLK,)
    return pl.pallas_call(
        _dense_body,
        grid=grid,
        in_specs=[
            pl.BlockSpec((2, _BLK, H), lambda i: (0, i, 0)),
            pl.BlockSpec((2, _BLK, H), lambda i: (0, i, 0)),
            pl.BlockSpec((D, H), lambda i: (0, 0)),
            pl.BlockSpec((D, H), lambda i: (0, 0)),
            pl.BlockSpec((1, D), lambda i: (0, 0)),
            pl.BlockSpec((D, H), lambda i: (0, 0)),
            pl.BlockSpec((D, H), lambda i: (0, 0)),
            pl.BlockSpec((1, D), lambda i: (0, 0)),
        ],
        out_specs=[
            pl.BlockSpec((_BLK, D), lambda i: (i, 0)),
            pl.BlockSpec((2, _BLK, H), lambda i: (0, i, 0)),
        ],
        out_shape=[
            jax.ShapeDtypeStruct((N, D), jnp.float32),
            jax.ShapeDtypeStruct((2, N, H), jnp.float32),
        ],
    )(e2, sp2, w1[:, :H], w1[:, H:], b1.reshape(1, D),
      w2[:, :H], w2[:, H:], b2.reshape(1, D))


def _loss_body(bu_ref, bp_ref, bn_ref, out_ref):
    bu = bu_ref[...]
    pos = jnp.sum(bu * bp_ref[...], axis=1)
    neg = jnp.sum(bu * bn_ref[...], axis=1)
    x = pos - neg
    out_ref[0, 0] = -jnp.mean(jnp.log(jax.nn.sigmoid(x)))


def _loss(bu, bp, bn):
    return pl.pallas_call(
        _loss_body,
        out_shape=jax.ShapeDtypeStruct((1, 1), jnp.float32),
        out_specs=pl.BlockSpec(memory_space=pltpu.SMEM),
    )(bu, bp, bn)[0, 0]


def kernel(u, i, j, edge_index, edge_vals, user_emb, item_emb, W1_w, W1_b, W2_w, W2_b):
    emb = jnp.concatenate((user_emb, item_emb), axis=0)
    e2 = jnp.stack([emb[:, :H], emb[:, H:]])
    pad = _EPAD - E
    src = jnp.concatenate([edge_index[1].astype(jnp.int32),
                           jnp.zeros((pad,), jnp.int32)]).reshape(_EROWS, 128)
    dst = jnp.concatenate([edge_index[0].astype(jnp.int32),
                           jnp.zeros((pad,), jnp.int32)]).reshape(_EROWS, 128)
    val = jnp.concatenate([edge_vals.astype(jnp.float32),
                           jnp.zeros((pad,), jnp.float32)])

    finals = [emb]
    for l in range(L):
        sp2 = _spmm_call(src, dst, val, e2.reshape(2 * N, H))
        norm, e2 = _dense_layer(e2, sp2, W1_w[l], W1_b[l], W2_w[l], W2_b[l])
        finals.append(norm)

    final = jnp.concatenate(finals, axis=1)
    bu = jnp.take(final[:N_USERS], u, axis=0)
    bp = jnp.take(final[N_USERS:], i, axis=0)
    bn = jnp.take(final[N_USERS:], j, axis=0)
    return _loss(bu, bp, bn)
